# Initial kernel scaffold; baseline (speedup 1.0000x reference)
import jax
import jax.numpy as jnp
from jax.experimental import pallas as pl


def kernel(query_points, feature_grid):
    raise NotImplementedError("write your pallas kernel here")


# speedup vs baseline: 1.3037x; 1.3037x over previous
import functools

import jax
import jax.numpy as jnp
from jax import lax
from jax.experimental import pallas as pl
from jax.experimental.pallas import tpu as pltpu, tpu_sc as plsc

_B, _C = 4, 32
_D = _H = _W = 64
_DHW = _D * _H * _W
_N = 100000
_NQ = _B * _N
_G = 160
_NCHUNK = _NQ // _G
_CHUNKS_PER_BATCH = _N // _G
_NCORES, _NSUB = 2, 16
_NW = _NCORES * _NSUB
_K = 8
_IDX_PER_CHUNK = _K * _G
_IDX_PER_DMA = 128
_NDMA = _IDX_PER_CHUNK // _IDX_PER_DMA

def _rsqrt16(v):
    i = plsc.bitcast(v, jnp.int32)
    magic = jnp.full((16,), 0x5F3759DF, jnp.int32)
    y = plsc.bitcast(magic - (i >> 1), jnp.float32)
    half = v * 0.5
    for _ in range(3):
        y = y * (1.5 - half * y * y)
    return y


def _sc_body(table_hbm, qp_hbm, out_hbm, qp_v, idx_v, wgt_v, rows_v, out_v, sem):
    wid = lax.axis_index("s") * _NCORES + lax.axis_index("c")
    niter = (_NCHUNK - wid + _NW - 1) // _NW
    lanes = jnp.arange(16, dtype=jnp.int32)

    def chunk_body(t, _):
        c = wid + t * _NW
        pltpu.sync_copy(qp_hbm.at[pl.ds(c * (_G * 3), _G * 3)], qp_v)
        batch_off = (c // _CHUNKS_PER_BATCH) * _DHW

        for g in range(_G // 16):
            pos3 = (g * 16) * 3 + lanes * 3
            gx = plsc.load_gather(qp_v, [pos3])
            gy = plsc.load_gather(qp_v, [pos3 + 1])
            gz = plsc.load_gather(qp_v, [pos3 + 2])
            ix = jnp.clip((gx + 1.0) * (0.5 * (_W - 1)), 0.0, _W - 1.0)
            iy = jnp.clip((gy + 1.0) * (0.5 * (_H - 1)), 0.0, _H - 1.0)
            iz = jnp.clip((gz + 1.0) * (0.5 * (_D - 1)), 0.0, _D - 1.0)
            x0 = jnp.minimum(ix.astype(jnp.int32), _W - 2)
            y0 = jnp.minimum(iy.astype(jnp.int32), _H - 2)
            z0 = jnp.minimum(iz.astype(jnp.int32), _D - 2)
            wx = ix - x0.astype(jnp.float32)
            wy = iy - y0.astype(jnp.float32)
            wz = iz - z0.astype(jnp.float32)
            ux = 1.0 - wx
            uy = 1.0 - wy
            uz = 1.0 - wz
            base = batch_off + z0 * (_H * _W) + y0 * _W + x0
            a00 = uz * uy
            a01 = uz * wy
            a10 = wz * uy
            a11 = wz * wy
            offs = (0, 1, _W, _W + 1, _H * _W, _H * _W + 1,
                    _H * _W + _W, _H * _W + _W + 1)
            wgts = (a00 * ux, a00 * wx, a01 * ux, a01 * wx,
                    a10 * ux, a10 * wx, a11 * ux, a11 * wx)
            for k in range(_K):
                p0 = k * _G + g * 16
                row, col = p0 // 128, p0 % 128
                idx_v[row, pl.ds(col, 16)] = base + offs[k]
                wgt_v[pl.ds(p0, 16)] = wgts[k]

        copies = [
            pltpu.async_copy(
                table_hbm.at[idx_v.at[j]],
                rows_v.at[pl.ds(j * _IDX_PER_DMA, _IDX_PER_DMA)],
                sem,
            )
            for j in range(_NDMA)
        ]
        for cp in copies:
            cp.wait()

        def q_body(q, _):
            acc0 = jnp.zeros((16,), jnp.float32)
            acc1 = jnp.zeros((16,), jnp.float32)
            for k in range(_K):
                p = k * _G + q
                w = plsc.load_gather(wgt_v, [jnp.full((16,), k * _G, jnp.int32) + q])
                acc0 = acc0 + w * rows_v[p, pl.ds(0, 16)]
                acc1 = acc1 + w * rows_v[p, pl.ds(16, 16)]
            ss = jnp.sum(acc0 * acc0 + acc1 * acc1)
            sv = jnp.maximum(jnp.broadcast_to(ss, (16,)), 1e-14)
            r = _rsqrt16(sv)
            out_v[q, pl.ds(0, 16)] = acc0 * r
            out_v[q, pl.ds(16, 16)] = acc1 * r
            return _

        lax.fori_loop(0, _G, q_body, None)

        pltpu.sync_copy(out_v, out_hbm.at[pl.ds(c * _G, _G)])
        return _

    lax.fori_loop(0, niter, chunk_body, None)


@functools.partial(
    pl.kernel,
    out_type=jax.ShapeDtypeStruct((_NQ, _C), jnp.float32),
    mesh=plsc.VectorSubcoreMesh(core_axis_name="c", subcore_axis_name="s"),
    compiler_params=pltpu.CompilerParams(
        needs_layout_passes=False, use_tc_tiling_on_sc=False
    ),
    scratch_types=[
        pltpu.VMEM((_G * 3,), jnp.float32),
        pltpu.VMEM((_NDMA, _IDX_PER_DMA), jnp.int32),
        pltpu.VMEM((_IDX_PER_CHUNK,), jnp.float32),
        pltpu.VMEM((_IDX_PER_CHUNK, _C), jnp.float32),
        pltpu.VMEM((_G, _C), jnp.float32),
        pltpu.SemaphoreType.DMA,
    ],
)
def _grid_decode_sc(table_hbm, qp_hbm, out_hbm, *scratch):
    _sc_body(table_hbm, qp_hbm, out_hbm, *scratch)


def kernel(query_points, feature_grid):
    table = jnp.transpose(feature_grid, (0, 2, 3, 4, 1)).reshape(_B * _DHW, _C)
    qp = query_points.reshape(_NQ * 3)
    out = _grid_decode_sc(table, qp)
    return out.reshape(_B, _N, _C)
